# baseline (speedup 1.0000x reference)
import functools

import jax
import jax.numpy as jnp
from jax import lax
from jax.experimental import pallas as pl

_KU = (0x6D3E048F, 0x1022172D)
_KS = (0x03D7B32D, 0xADD083F4)

_EPS = 1e-12
_TINY = float(jnp.finfo(jnp.float32).tiny)
_ROT = ((13, 15, 26, 6), (17, 29, 16, 24))


def _threefry_bits(cnt, k0, k1):
    ks = (k0, k1, k0 ^ k1 ^ 0x1BD11BDA)
    x0 = jnp.full(cnt.shape, jnp.uint32(k0))
    x1 = cnt + jnp.uint32(k1)
    for b in range(5):
        for r in _ROT[b % 2]:
            x0 = x0 + x1
            x1 = (x1 << jnp.uint32(r)) | (x1 >> jnp.uint32(32 - r))
            x1 = x1 ^ x0
        x0 = x0 + jnp.uint32(ks[(b + 1) % 3])
        x1 = x1 + jnp.uint32((ks[(b + 2) % 3] + b + 1) & 0xFFFFFFFF)
    return x0 ^ x1


def _bits_to_unit_float(bits):
    f = lax.bitcast_convert_type(
        (bits >> jnp.uint32(9)) | jnp.uint32(0x3F800000), jnp.float32
    )
    return f - jnp.float32(1.0)


def _stgs_body(x_ref, y_ref, ids_ref, *, rows_per_step, vocab):
    i = pl.program_id(0)
    x = x_ref[...]
    shape = (rows_per_step, vocab)
    col = lax.broadcasted_iota(jnp.int32, shape, 1)
    row = lax.broadcasted_iota(jnp.int32, shape, 0) + i * rows_per_step
    cnt = (row * vocab + col).astype(jnp.uint32)

    f = _bits_to_unit_float(_threefry_bits(cnt, *_KU))
    u = f * jnp.float32(0.999 - _EPS) + jnp.float32(_EPS)
    gl = x + (-jnp.log(-jnp.log(u)))

    m = jnp.max(gl, axis=-1, keepdims=True)
    e = jnp.exp(gl - m)
    y = e / jnp.sum(e, axis=-1, keepdims=True)
    y_ref[...] = y

    f2 = _bits_to_unit_float(_threefry_bits(cnt, *_KS))
    u2 = jnp.maximum(
        f2 * jnp.float32(1.0 - _TINY) + jnp.float32(_TINY), jnp.float32(_TINY)
    )
    vals = jnp.log(y + jnp.float32(_EPS)) + (-jnp.log(-jnp.log(u2)))

    vm = jnp.max(vals, axis=-1, keepdims=True)
    idx = jnp.min(
        jnp.where(vals == vm, col, jnp.int32(vocab)), axis=-1
    )
    ids_ref[...] = jnp.broadcast_to(idx[:, None], (rows_per_step, 128))


@functools.partial(jax.jit, static_argnames=("interpret",))
def kernel(x, interpret=False):
    b0, b1, vocab = x.shape
    rows = b0 * b1
    rows_per_step = 8 if rows % 8 == 0 else rows
    grid = rows // rows_per_step
    x2 = x.reshape(rows, vocab)

    y2, ids2 = pl.pallas_call(
        functools.partial(
            _stgs_body, rows_per_step=rows_per_step, vocab=vocab
        ),
        grid=(grid,),
        in_specs=[
            pl.BlockSpec((rows_per_step, vocab), lambda i: (i, 0)),
        ],
        out_specs=[
            pl.BlockSpec((rows_per_step, vocab), lambda i: (i, 0)),
            pl.BlockSpec((rows_per_step, 128), lambda i: (i, 0)),
        ],
        out_shape=[
            jax.ShapeDtypeStruct((rows, vocab), jnp.float32),
            jax.ShapeDtypeStruct((rows, 128), jnp.int32),
        ],
        interpret=interpret,
    )(x2)

    output_ids = ids2[:, 0].reshape(b0, b1)
    y_soft = y2.reshape(b0, b1, vocab)
    eff_temperature = jnp.asarray([1.0], dtype=x.dtype)
    return output_ids, y_soft, eff_temperature
